# baseline (speedup 1.0000x reference)
import jax
import jax.numpy as jnp
from jax import lax
from jax.experimental import pallas as pl
from jax.experimental.pallas import tpu as pltpu
from jax.experimental.pallas import tpu_sc as plsc

N = 100000
D = 128
S = 512
NC = 2
NS = 16
NW = NC * NS
G = 16
CW = 3120
CL = 3280
TR = 240
NT = CW // TR
GPT = TR // G
TAILR = CL - CW
L = 16
NV = D // L

_NEG_INF = float("-inf")


def _sc_body(x_hbm, batch_hbm, out_hbm, bids_hbm, bvals_hbm,
             bbuf, xb0, xb1, st0, st1, cbuf, ibuf, nbuf, acc_ref,
             sem0, sem1, semf0, semf1):
    wid = lax.axis_index("s") * NC + lax.axis_index("c")
    base = wid * CW

    pltpu.sync_copy(batch_hbm.at[pl.ds(base, CL)], bbuf.at[pl.ds(0, CL)])
    neg = jnp.full((L,), _NEG_INF, jnp.float32)
    for d in range(NV):
        cbuf[pl.ds(d * L, L)] = neg
        acc_ref[pl.ds(d * L, L)] = neg

    first_id = bbuf[pl.ds(0, L)][0]

    def flush(cur, k, acc):
        @pl.when(k == 0)
        def _():
            for d in range(NV):
                st0[pl.ds(d * L, L)] = acc[d]
            pltpu.sync_copy(st0, bvals_hbm.at[wid, 0])

        @pl.when(jnp.logical_and(k != 0, k % 2 == 1))
        def _():
            @pl.when(k >= 3)
            def _():
                pltpu.make_async_copy(out_hbm.at[0], st1, semf1).wait()
            for d in range(NV):
                st1[pl.ds(d * L, L)] = acc[d]
            pltpu.async_copy(st1, out_hbm.at[cur], semf1)

        @pl.when(jnp.logical_and(k != 0, k % 2 == 0))
        def _():
            @pl.when(k >= 4)
            def _():
                pltpu.make_async_copy(out_hbm.at[0], st0, semf0).wait()
            for d in range(NV):
                st0[pl.ds(d * L, L)] = acc[d]
            pltpu.async_copy(st0, out_hbm.at[cur], semf0)

    def gap_fill(cur, nxt):
        def body(s, carry):
            pltpu.sync_copy(cbuf, out_hbm.at[s])
            return carry
        lax.fori_loop(cur + 1, nxt, body, 0)

    def make_group_body(xref, toff):
        def group_body(g, carry):
            goff = toff + g * G
            lrow = g * G
            idvec = bbuf[pl.ds(goff, L)]
            cur0 = carry[0]

            def fast(carry):
                acc = [acc_ref[pl.ds(d * L, L)] for d in range(NV)]
                for r in range(G):
                    for d in range(NV):
                        acc[d] = jnp.maximum(
                            acc[d], xref[lrow + r, pl.ds(d * L, L)])
                for d in range(NV):
                    acc_ref[pl.ds(d * L, L)] = acc[d]
                return carry

            def slow(carry):
                def row_step(r, rcarry):
                    cur, k = rcarry[0], rcarry[1]
                    acc = rcarry[2:]
                    b = bbuf[pl.ds(goff + r, L)][0]
                    changed = b != cur

                    @pl.when(changed)
                    def _():
                        flush(cur, k, acc)
                        gap_fill(cur, b)

                    row = tuple(
                        xref[lrow + r, pl.ds(d * L, L)] for d in range(NV))
                    new_acc = tuple(
                        jnp.where(changed, rd, jnp.maximum(ad, rd))
                        for ad, rd in zip(acc, row)
                    )
                    return (b, k + changed.astype(jnp.int32)) + new_acc

                acc0 = tuple(acc_ref[pl.ds(d * L, L)] for d in range(NV))
                out = lax.fori_loop(0, G, row_step, carry + acc0)
                for d in range(NV):
                    acc_ref[pl.ds(d * L, L)] = out[2 + d]
                return (out[0], out[1])

            return lax.cond(idvec[L - 1] == cur0, fast, slow, carry)
        return group_body

    carry = (first_id, jnp.int32(0))

    xbufs = (xb0, xb1)
    sems = (sem0, sem1)
    copies = [None, None]
    copies[0] = pltpu.async_copy(x_hbm.at[pl.ds(base, TR), :], xb0, sem0)
    for t in range(NT):
        if t + 1 < NT:
            nb = (t + 1) % 2
            copies[nb] = pltpu.async_copy(
                x_hbm.at[pl.ds(base + (t + 1) * TR, TR), :], xbufs[nb],
                sems[nb])
        copies[t % 2].wait()
        carry = lax.fori_loop(
            0, GPT, make_group_body(xbufs[t % 2], t * TR), carry)

    def tail(carry):
        pltpu.sync_copy(
            x_hbm.at[pl.ds(base + CW, TAILR), :], xb0.at[pl.ds(0, TAILR), :])
        return lax.fori_loop(
            0, TAILR // G, make_group_body(xb0, CW), carry)

    carry = lax.cond(wid == NW - 1, tail, lambda c: c, carry)
    cur, k = carry[0], carry[1]
    acc = tuple(acc_ref[pl.ds(d * L, L)] for d in range(NV))

    @pl.when(k >= 2)
    def _():
        pltpu.make_async_copy(out_hbm.at[0], st1, semf1).wait()

    @pl.when(k >= 3)
    def _():
        pltpu.make_async_copy(out_hbm.at[0], st0, semf0).wait()

    for d in range(NV):
        st1[pl.ds(d * L, L)] = acc[d]
    pltpu.sync_copy(st1, bvals_hbm.at[wid, 1])

    @pl.when(k == 0)
    def _():
        pltpu.sync_copy(st1, bvals_hbm.at[wid, 0])

    @pl.when(wid == 0)
    def _():
        gap_fill(jnp.int32(-1), first_id)

    @pl.when(wid < NW - 1)
    def _():
        pltpu.sync_copy(batch_hbm.at[pl.ds((wid + 1) * CW, L)], nbuf)
        nxt_lo = nbuf[...][0]
        gap_fill(cur, nxt_lo)

    @pl.when(wid == NW - 1)
    def _():
        gap_fill(cur, jnp.int32(S))

    ids = lax.iota(jnp.int32, L)
    idvec = jnp.where(ids == 0, first_id, jnp.where(ids == 1, cur, 0))
    ibuf[...] = idvec
    pltpu.sync_copy(ibuf, bids_hbm.at[wid])


_sc_pool = pl.kernel(
    _sc_body,
    out_type=(
        jax.ShapeDtypeStruct((S, D), jnp.float32),
        jax.ShapeDtypeStruct((NW, L), jnp.int32),
        jax.ShapeDtypeStruct((NW, 2, D), jnp.float32),
    ),
    mesh=plsc.VectorSubcoreMesh(core_axis_name="c", subcore_axis_name="s"),
    compiler_params=pltpu.CompilerParams(use_tc_tiling_on_sc=False),
    scratch_types=[
        pltpu.VMEM((CL + G,), jnp.int32),
        pltpu.VMEM((TR, D), jnp.float32),
        pltpu.VMEM((TR, D), jnp.float32),
        pltpu.VMEM((D,), jnp.float32),
        pltpu.VMEM((D,), jnp.float32),
        pltpu.VMEM((D,), jnp.float32),
        pltpu.VMEM((L,), jnp.int32),
        pltpu.VMEM((L,), jnp.int32),
        pltpu.VMEM((D,), jnp.float32),
        pltpu.SemaphoreType.DMA,
        pltpu.SemaphoreType.DMA,
        pltpu.SemaphoreType.DMA,
        pltpu.SemaphoreType.DMA,
    ],
)


def _merge_body(bids_smem, interior, bvals, out_ref):
    out_ref[...] = interior[...]
    neg_row = jnp.full((1, D), _NEG_INF, jnp.float32)
    for w in range(NW):
        for j in range(2):
            q = bids_smem[w, j]
            out_ref[pl.ds(q, 1), :] = neg_row
    for w in range(NW):
        for j in range(2):
            q = bids_smem[w, j]
            row = bvals[w, j, :].reshape(1, D)
            out_ref[pl.ds(q, 1), :] = jnp.maximum(out_ref[pl.ds(q, 1), :], row)


_merge = pl.pallas_call(
    _merge_body,
    out_shape=jax.ShapeDtypeStruct((S, D), jnp.float32),
    in_specs=[
        pl.BlockSpec(memory_space=pltpu.SMEM),
        pl.BlockSpec(memory_space=pltpu.VMEM),
        pl.BlockSpec(memory_space=pltpu.VMEM),
    ],
)


@jax.jit
def kernel(x, batch):
    interior, bids, bvals = _sc_pool(x, batch)
    return _merge(bids, interior, bvals)
